# Initial kernel scaffold; baseline (speedup 1.0000x reference)
import functools

import jax
import jax.numpy as jnp
from jax import lax
from jax.experimental import pallas as pl
from jax.experimental.pallas import tpu as pltpu
from jax.experimental.pallas import tpu_sc as plsc

NC, NS = 2, 16
NW = NC * NS
CH = 128
NUM_GRAPHS = 64
BLK = 512


@functools.lru_cache(maxsize=None)
def _deg_kernel(n_acc, k_chunks):
  mesh = plsc.VectorSubcoreMesh(core_axis_name="c", subcore_axis_name="s")
  stripe = n_acc // NS

  @functools.partial(
      pl.kernel,
      mesh=mesh,
      out_type=jax.ShapeDtypeStruct((NC, n_acc, 8), jnp.float32),
      scratch_types=[
          pltpu.VMEM((k_chunks, CH), jnp.int32),
          pltpu.VMEM((CH, 8), jnp.float32),
          pltpu.VMEM_SHARED((n_acc, 8), jnp.float32),
      ],
  )
  def deg(dst_hbm, ones_hbm, zero_hbm, dp_out, dst_v, ones_v, acc):
    c = lax.axis_index("c")
    s = lax.axis_index("s")
    wid = c * NS + s
    pltpu.sync_copy(dst_hbm.at[wid], dst_v)
    pltpu.sync_copy(ones_hbm, ones_v)
    pltpu.sync_copy(zero_hbm.at[pl.ds(s * stripe, stripe)],
                    acc.at[pl.ds(s * stripe, stripe)])
    plsc.subcore_barrier()

    def body(j, carry):
      pltpu.sync_copy(ones_v, acc.at[dst_v.at[j]], add=True)
      return carry

    lax.fori_loop(0, k_chunks, body, 0)
    plsc.subcore_barrier()
    pltpu.sync_copy(acc.at[pl.ds(s * stripe, stripe)],
                    dp_out.at[c, pl.ds(s * stripe, stripe)])

  return deg


@functools.lru_cache(maxsize=None)
def _edge_kernel(n_acc, h, k_chunks):
  mesh = plsc.VectorSubcoreMesh(core_axis_name="c", subcore_axis_name="s")
  stripe = n_acc // NS

  @functools.partial(
      pl.kernel,
      mesh=mesh,
      out_type=jax.ShapeDtypeStruct((NC, n_acc, h), jnp.float32),
      scratch_types=[
          pltpu.VMEM((k_chunks, CH), jnp.int32),
          pltpu.VMEM((k_chunks, CH), jnp.int32),
          pltpu.VMEM((2, CH, h), jnp.float32),
          pltpu.VMEM_SHARED((n_acc, h), jnp.float32),
          pltpu.SemaphoreType.DMA,
          pltpu.SemaphoreType.DMA,
      ],
  )
  def edge(h_hbm, src_hbm, dst_hbm, zero_hbm, p_out,
           src_v, dst_v, rows_v, acc, sem0, sem1):
    c = lax.axis_index("c")
    s = lax.axis_index("s")
    wid = c * NS + s
    pltpu.sync_copy(src_hbm.at[wid], src_v)
    pltpu.sync_copy(dst_hbm.at[wid], dst_v)
    pltpu.sync_copy(zero_hbm.at[pl.ds(s * stripe, stripe)],
                    acc.at[pl.ds(s * stripe, stripe)])
    plsc.subcore_barrier()

    sems = (sem0, sem1)

    def gather(j, b):
      return pltpu.make_async_copy(h_hbm.at[src_v.at[j]], rows_v.at[b],
                                   sems[b])

    gather(0, 0).start()

    def body(i, carry):
      j0 = 2 * i
      j1 = j0 + 1
      gather(j0, 0).wait()
      gather(j1, 1).start()
      pltpu.sync_copy(rows_v.at[0], acc.at[dst_v.at[j0]], add=True)
      gather(j1, 1).wait()

      @pl.when(i < k_chunks // 2 - 1)
      def _():
        gather(j1 + 1, 0).start()

      pltpu.sync_copy(rows_v.at[1], acc.at[dst_v.at[j1]], add=True)
      return carry

    lax.fori_loop(0, k_chunks // 2, body, 0)
    plsc.subcore_barrier()
    pltpu.sync_copy(acc.at[pl.ds(s * stripe, stripe)],
                    p_out.at[c, pl.ds(s * stripe, stripe)])

  return edge


@functools.lru_cache(maxsize=None)
def _pool_kernel(n_acc, h, g):
  mesh = plsc.VectorSubcoreMesh(core_axis_name="c", subcore_axis_name="s")
  rows = n_acc // NW
  gp = g + 1

  @functools.partial(
      pl.kernel,
      mesh=mesh,
      out_type=(
          jax.ShapeDtypeStruct((NW, gp, h), jnp.float32),
          jax.ShapeDtypeStruct((NW, gp, h), jnp.float32),
          jax.ShapeDtypeStruct((NW, gp, h), jnp.float32),
      ),
      scratch_types=[
          pltpu.VMEM((rows, h), jnp.float32),
          pltpu.VMEM((rows,), jnp.int32),
          pltpu.VMEM((gp, h), jnp.float32),
          pltpu.VMEM((gp, h), jnp.float32),
          pltpu.VMEM((gp, h), jnp.float32),
      ],
  )
  def pool(hid_hbm, bat_hbm, neg_hbm, zero_hbm, sums_o, maxs_o, cnts_o,
           hid_v, bat_v, sum_v, max_v, cnt_v):
    c = lax.axis_index("c")
    s = lax.axis_index("s")
    wid = c * NS + s
    pltpu.sync_copy(hid_hbm.at[pl.ds(wid * rows, rows)], hid_v)
    pltpu.sync_copy(bat_hbm.at[wid], bat_v)
    pltpu.sync_copy(neg_hbm, max_v)
    pltpu.sync_copy(zero_hbm.at[pl.ds(0, gp)], sum_v)
    pltpu.sync_copy(zero_hbm.at[pl.ds(0, gp)], cnt_v)

    def body(i, carry):
      b = bat_v[i]
      for k in range(h // 16):
        sl = pl.ds(k * 16, 16)
        r = hid_v[i, sl]
        sum_v[b, sl] = sum_v[b, sl] + r
        max_v[b, sl] = jnp.maximum(max_v[b, sl], r)
      c16 = pl.ds(0, 16)
      cnt_v[b, c16] = cnt_v[b, c16] + 1.0
      return carry

    lax.fori_loop(0, rows, body, 0)
    pltpu.sync_copy(sum_v, sums_o.at[wid])
    pltpu.sync_copy(max_v, maxs_o.at[wid])
    pltpu.sync_copy(cnt_v, cnts_o.at[wid])

  return pool


def _tc1_body(x_ref, w_ref, dp_ref, h_out, dinv_out):
  dp = dp_ref[:]
  dinv = lax.rsqrt(dp[0] + dp[1] + 1.0)
  hm = jnp.dot(x_ref[:], w_ref[:], preferred_element_type=jnp.float32)
  h_out[:] = hm * dinv[:, :1]
  dinv_out[:] = dinv


def _tc1(x_pad, w1, dp):
  n_acc, d = x_pad.shape
  h = w1.shape[1]
  grid = n_acc // BLK
  return pl.pallas_call(
      _tc1_body,
      grid=(grid,),
      in_specs=[
          pl.BlockSpec((BLK, d), lambda i: (i, 0)),
          pl.BlockSpec((d, h), lambda i: (0, 0)),
          pl.BlockSpec((NC, BLK, 8), lambda i: (0, i, 0)),
      ],
      out_specs=[
          pl.BlockSpec((BLK, h), lambda i: (i, 0)),
          pl.BlockSpec((BLK, 8), lambda i: (i, 0)),
      ],
      out_shape=[
          jax.ShapeDtypeStruct((n_acc, h), jnp.float32),
          jax.ShapeDtypeStruct((n_acc, 8), jnp.float32),
      ],
  )(x_pad, w1, dp)


def _tc_mid_body(p_ref, hp_ref, dinv_ref, b_ref, w_ref, out_ref):
  p = p_ref[:]
  d = dinv_ref[:, :1]
  t = jnp.tanh(d * (p[0] + p[1] + hp_ref[:]) + b_ref[:])
  out_ref[:] = jnp.dot(t, w_ref[:], preferred_element_type=jnp.float32) * d


def _tc_mid(p, hp, dinv8, b, w):
  n_acc, h = hp.shape
  grid = n_acc // BLK
  return pl.pallas_call(
      _tc_mid_body,
      grid=(grid,),
      in_specs=[
          pl.BlockSpec((NC, BLK, h), lambda i: (0, i, 0)),
          pl.BlockSpec((BLK, h), lambda i: (i, 0)),
          pl.BlockSpec((BLK, 8), lambda i: (i, 0)),
          pl.BlockSpec((1, h), lambda i: (0, 0)),
          pl.BlockSpec((h, h), lambda i: (0, 0)),
      ],
      out_specs=pl.BlockSpec((BLK, h), lambda i: (i, 0)),
      out_shape=jax.ShapeDtypeStruct((n_acc, h), jnp.float32),
  )(p, hp, dinv8, b, w)


def _tc_last_body(p_ref, hp_ref, dinv_ref, b_ref, out_ref):
  p = p_ref[:]
  d = dinv_ref[:, :1]
  out_ref[:] = jnp.tanh(d * (p[0] + p[1] + hp_ref[:]) + b_ref[:])


def _tc_last(p, hp, dinv8, b):
  n_acc, h = hp.shape
  grid = n_acc // BLK
  return pl.pallas_call(
      _tc_last_body,
      grid=(grid,),
      in_specs=[
          pl.BlockSpec((NC, BLK, h), lambda i: (0, i, 0)),
          pl.BlockSpec((BLK, h), lambda i: (i, 0)),
          pl.BlockSpec((BLK, 8), lambda i: (i, 0)),
          pl.BlockSpec((1, h), lambda i: (0, 0)),
      ],
      out_specs=pl.BlockSpec((BLK, h), lambda i: (i, 0)),
      out_shape=jax.ShapeDtypeStruct((n_acc, h), jnp.float32),
  )(p, hp, dinv8, b)


def _tc_final_body(sums_ref, maxs_ref, cnts_ref, wfc_ref, bfc_ref, out_ref):
  g = NUM_GRAPHS
  s = jnp.sum(sums_ref[:], axis=0)[:g]
  m = jnp.max(maxs_ref[:], axis=0)[:g]
  cnt = jnp.sum(cnts_ref[:], axis=0)[:g, :1]
  gmean = s / jnp.maximum(cnt, 1.0)
  pooled = jnp.concatenate([m, gmean], axis=1)
  out_ref[:] = (
      jnp.dot(pooled, wfc_ref[:], preferred_element_type=jnp.float32)
      + bfc_ref[:])


def _tc_final(sums, maxs, cnts, wfc, bfc):
  c = wfc.shape[1]
  return pl.pallas_call(
      _tc_final_body,
      out_shape=jax.ShapeDtypeStruct((NUM_GRAPHS, c), jnp.float32),
  )(sums, maxs, cnts, wfc, bfc)


def kernel(x, edge_index, batch_index, W1, b1, W2, b2, W3, b3, W4, b4,
           Wfc, bfc):
  n, d = x.shape
  h = W1.shape[1]
  e = edge_index.shape[1]
  g = NUM_GRAPHS

  n_acc = pl.cdiv(n + 1, BLK) * BLK
  k_chunks = pl.cdiv(e, NW * CH)
  k_chunks += k_chunks % 2
  e_pad = NW * k_chunks * CH

  src = edge_index[0].astype(jnp.int32)
  dst = edge_index[1].astype(jnp.int32)
  src_r = jnp.concatenate(
      [src, jnp.full((e_pad - e,), n, jnp.int32)]).reshape(NW, k_chunks, CH)
  dst_r = jnp.concatenate(
      [dst, jnp.full((e_pad - e,), n, jnp.int32)]).reshape(NW, k_chunks, CH)
  x_pad = jnp.pad(x, ((0, n_acc - n), (0, 0)))
  bat_r = jnp.pad(batch_index.astype(jnp.int32), (0, n_acc - n),
                  constant_values=g).reshape(NW, n_acc // NW)

  zero_h = jnp.zeros((n_acc, h), jnp.float32)
  zero_8 = jnp.zeros((n_acc, 8), jnp.float32)
  ones_8 = jnp.ones((CH, 8), jnp.float32)
  neg = jnp.full((g + 1, h), -jnp.inf, jnp.float32)

  dp = _deg_kernel(n_acc, k_chunks)(dst_r, ones_8, zero_8)
  h1p, dinv8 = _tc1(x_pad, W1, dp)

  edge = _edge_kernel(n_acc, h, k_chunks)
  p = edge(h1p, src_r, dst_r, zero_h)
  h2p = _tc_mid(p, h1p, dinv8, b1.reshape(1, h), W2)
  p = edge(h2p, src_r, dst_r, zero_h)
  h3p = _tc_mid(p, h2p, dinv8, b2.reshape(1, h), W3)
  p = edge(h3p, src_r, dst_r, zero_h)
  h4p = _tc_mid(p, h3p, dinv8, b3.reshape(1, h), W4)
  p = edge(h4p, src_r, dst_r, zero_h)
  hidden = _tc_last(p, h4p, dinv8, b4.reshape(1, h))

  sums, maxs, cnts = _pool_kernel(n_acc, h, g)(hidden, bat_r, neg, zero_h)
  out = _tc_final(sums, maxs, cnts, Wfc, bfc.reshape(1, Wfc.shape[1]))
  return (out, hidden[:n])


# speedup vs baseline: 10.9261x; 10.9261x over previous
import functools

import jax
import jax.numpy as jnp
from jax import lax
from jax.experimental import pallas as pl
from jax.experimental.pallas import tpu as pltpu
from jax.experimental.pallas import tpu_sc as plsc

NC, NS = 2, 16
NW = NC * NS
CH = 128
NUM_GRAPHS = 64
BLK = 512


@functools.lru_cache(maxsize=None)
def _deg_kernel(n_acc, k_chunks):
  mesh = plsc.VectorSubcoreMesh(core_axis_name="c", subcore_axis_name="s")
  stripe = n_acc // NS

  @functools.partial(
      pl.kernel,
      mesh=mesh,
      compiler_params=pltpu.CompilerParams(use_tc_tiling_on_sc=False),
      out_type=jax.ShapeDtypeStruct((NC, n_acc, 8), jnp.float32),
      scratch_types=[
          pltpu.VMEM((k_chunks, CH), jnp.int32),
          pltpu.VMEM((CH, 8), jnp.float32),
          pltpu.VMEM_SHARED((n_acc, 8), jnp.float32),
      ],
  )
  def deg(dst_hbm, ones_hbm, zero_hbm, dp_out, dst_v, ones_v, acc):
    c = lax.axis_index("c")
    s = lax.axis_index("s")
    wid = c * NS + s
    pltpu.sync_copy(dst_hbm.at[wid], dst_v)
    pltpu.sync_copy(ones_hbm, ones_v)
    pltpu.sync_copy(zero_hbm.at[pl.ds(s * stripe, stripe)],
                    acc.at[pl.ds(s * stripe, stripe)])
    plsc.subcore_barrier()

    def body(j, carry):
      pltpu.sync_copy(ones_v, acc.at[dst_v.at[j]], add=True)
      return carry

    lax.fori_loop(0, k_chunks, body, 0)
    plsc.subcore_barrier()
    pltpu.sync_copy(acc.at[pl.ds(s * stripe, stripe)],
                    dp_out.at[c, pl.ds(s * stripe, stripe)])

  return deg


@functools.lru_cache(maxsize=None)
def _edge_kernel(n_acc, h, k_chunks):
  mesh = plsc.VectorSubcoreMesh(core_axis_name="c", subcore_axis_name="s")
  stripe = n_acc // NS

  @functools.partial(
      pl.kernel,
      mesh=mesh,
      compiler_params=pltpu.CompilerParams(use_tc_tiling_on_sc=False),
      out_type=jax.ShapeDtypeStruct((NC, n_acc, h), jnp.float32),
      scratch_types=[
          pltpu.VMEM((k_chunks, CH), jnp.int32),
          pltpu.VMEM((k_chunks, CH), jnp.int32),
          pltpu.VMEM((2, CH, h), jnp.float32),
          pltpu.VMEM_SHARED((n_acc, h), jnp.float32),
          pltpu.SemaphoreType.DMA,
          pltpu.SemaphoreType.DMA,
      ],
  )
  def edge(h_hbm, src_hbm, dst_hbm, zero_hbm, p_out,
           src_v, dst_v, rows_v, acc, sem0, sem1):
    c = lax.axis_index("c")
    s = lax.axis_index("s")
    wid = c * NS + s
    pltpu.sync_copy(src_hbm.at[wid], src_v)
    pltpu.sync_copy(dst_hbm.at[wid], dst_v)
    pltpu.sync_copy(zero_hbm.at[pl.ds(s * stripe, stripe)],
                    acc.at[pl.ds(s * stripe, stripe)])
    plsc.subcore_barrier()

    sems = (sem0, sem1)

    def gather(j, b):
      return pltpu.make_async_copy(h_hbm.at[src_v.at[j]], rows_v.at[b],
                                   sems[b])

    gather(0, 0).start()

    def body(i, carry):
      j0 = 2 * i
      j1 = j0 + 1
      gather(j0, 0).wait()
      gather(j1, 1).start()
      pltpu.sync_copy(rows_v.at[0], acc.at[dst_v.at[j0]], add=True)
      gather(j1, 1).wait()

      @pl.when(i < k_chunks // 2 - 1)
      def _():
        gather(j1 + 1, 0).start()

      pltpu.sync_copy(rows_v.at[1], acc.at[dst_v.at[j1]], add=True)
      return carry

    lax.fori_loop(0, k_chunks // 2, body, 0)
    plsc.subcore_barrier()
    pltpu.sync_copy(acc.at[pl.ds(s * stripe, stripe)],
                    p_out.at[c, pl.ds(s * stripe, stripe)])

  return edge


@functools.lru_cache(maxsize=None)
def _pool_kernel(n_acc, h, g):
  mesh = plsc.VectorSubcoreMesh(core_axis_name="c", subcore_axis_name="s")
  rows = n_acc // NW
  gp = g + 1

  @functools.partial(
      pl.kernel,
      mesh=mesh,
      compiler_params=pltpu.CompilerParams(use_tc_tiling_on_sc=False),
      out_type=(
          jax.ShapeDtypeStruct((NW, gp, h), jnp.float32),
          jax.ShapeDtypeStruct((NW, gp, h), jnp.float32),
          jax.ShapeDtypeStruct((NW, gp, h), jnp.float32),
      ),
      scratch_types=[
          pltpu.VMEM((rows, h), jnp.float32),
          pltpu.VMEM((rows,), jnp.int32),
          pltpu.VMEM((gp, h), jnp.float32),
          pltpu.VMEM((gp, h), jnp.float32),
          pltpu.VMEM((gp, h), jnp.float32),
      ],
  )
  def pool(hid_hbm, bat_hbm, neg_hbm, zero_hbm, sums_o, maxs_o, cnts_o,
           hid_v, bat_v, sum_v, max_v, cnt_v):
    c = lax.axis_index("c")
    s = lax.axis_index("s")
    wid = c * NS + s
    pltpu.sync_copy(hid_hbm.at[pl.ds(wid * rows, rows)], hid_v)
    pltpu.sync_copy(bat_hbm.at[wid], bat_v)
    pltpu.sync_copy(neg_hbm, max_v)
    pltpu.sync_copy(zero_hbm.at[pl.ds(0, gp)], sum_v)
    pltpu.sync_copy(zero_hbm.at[pl.ds(0, gp)], cnt_v)

    def body(q, carry):
      base = q * 16
      bid = bat_v[pl.ds(base, 16)]
      for j in range(16):
        i = base + j
        b = bid[j]
        for k in range(h // 16):
          sl = pl.ds(k * 16, 16)
          r = hid_v[i, sl]
          sum_v[b, sl] = sum_v[b, sl] + r
          max_v[b, sl] = jnp.maximum(max_v[b, sl], r)
        c16 = pl.ds(0, 16)
        cnt_v[b, c16] = cnt_v[b, c16] + 1.0
      return carry

    lax.fori_loop(0, rows // 16, body, 0)
    pltpu.sync_copy(sum_v, sums_o.at[wid])
    pltpu.sync_copy(max_v, maxs_o.at[wid])
    pltpu.sync_copy(cnt_v, cnts_o.at[wid])

  return pool


def _tc1_body(x_ref, w_ref, dp_ref, h_out, dinv_out):
  dp = dp_ref[:]
  dinv = lax.rsqrt(dp[0] + dp[1] + 1.0)
  hm = jnp.dot(x_ref[:], w_ref[:], preferred_element_type=jnp.float32)
  h_out[:] = hm * dinv[:, :1]
  dinv_out[:] = dinv


def _tc1(x_pad, w1, dp):
  n_acc, d = x_pad.shape
  h = w1.shape[1]
  grid = n_acc // BLK
  return pl.pallas_call(
      _tc1_body,
      grid=(grid,),
      in_specs=[
          pl.BlockSpec((BLK, d), lambda i: (i, 0)),
          pl.BlockSpec((d, h), lambda i: (0, 0)),
          pl.BlockSpec((NC, BLK, 8), lambda i: (0, i, 0)),
      ],
      out_specs=[
          pl.BlockSpec((BLK, h), lambda i: (i, 0)),
          pl.BlockSpec((BLK, 8), lambda i: (i, 0)),
      ],
      out_shape=[
          jax.ShapeDtypeStruct((n_acc, h), jnp.float32),
          jax.ShapeDtypeStruct((n_acc, 8), jnp.float32),
      ],
  )(x_pad, w1, dp)


def _tc_mid_body(p_ref, hp_ref, dinv_ref, b_ref, w_ref, out_ref):
  p = p_ref[:]
  d = dinv_ref[:, :1]
  t = jnp.tanh(d * (p[0] + p[1] + hp_ref[:]) + b_ref[:])
  out_ref[:] = jnp.dot(t, w_ref[:], preferred_element_type=jnp.float32) * d


def _tc_mid(p, hp, dinv8, b, w):
  n_acc, h = hp.shape
  grid = n_acc // BLK
  return pl.pallas_call(
      _tc_mid_body,
      grid=(grid,),
      in_specs=[
          pl.BlockSpec((NC, BLK, h), lambda i: (0, i, 0)),
          pl.BlockSpec((BLK, h), lambda i: (i, 0)),
          pl.BlockSpec((BLK, 8), lambda i: (i, 0)),
          pl.BlockSpec((1, h), lambda i: (0, 0)),
          pl.BlockSpec((h, h), lambda i: (0, 0)),
      ],
      out_specs=pl.BlockSpec((BLK, h), lambda i: (i, 0)),
      out_shape=jax.ShapeDtypeStruct((n_acc, h), jnp.float32),
  )(p, hp, dinv8, b, w)


def _tc_last_body(p_ref, hp_ref, dinv_ref, b_ref, out_ref):
  p = p_ref[:]
  d = dinv_ref[:, :1]
  out_ref[:] = jnp.tanh(d * (p[0] + p[1] + hp_ref[:]) + b_ref[:])


def _tc_last(p, hp, dinv8, b):
  n_acc, h = hp.shape
  grid = n_acc // BLK
  return pl.pallas_call(
      _tc_last_body,
      grid=(grid,),
      in_specs=[
          pl.BlockSpec((NC, BLK, h), lambda i: (0, i, 0)),
          pl.BlockSpec((BLK, h), lambda i: (i, 0)),
          pl.BlockSpec((BLK, 8), lambda i: (i, 0)),
          pl.BlockSpec((1, h), lambda i: (0, 0)),
      ],
      out_specs=pl.BlockSpec((BLK, h), lambda i: (i, 0)),
      out_shape=jax.ShapeDtypeStruct((n_acc, h), jnp.float32),
  )(p, hp, dinv8, b)


def _tc_final_body(sums_ref, maxs_ref, cnts_ref, wfc_ref, bfc_ref, out_ref):
  g = NUM_GRAPHS
  s = jnp.sum(sums_ref[:], axis=0)[:g]
  m = jnp.max(maxs_ref[:], axis=0)[:g]
  cnt = jnp.sum(cnts_ref[:], axis=0)[:g, :1]
  gmean = s / jnp.maximum(cnt, 1.0)
  pooled = jnp.concatenate([m, gmean], axis=1)
  out_ref[:] = (
      jnp.dot(pooled, wfc_ref[:], preferred_element_type=jnp.float32)
      + bfc_ref[:])


def _tc_final(sums, maxs, cnts, wfc, bfc):
  c = wfc.shape[1]
  return pl.pallas_call(
      _tc_final_body,
      out_shape=jax.ShapeDtypeStruct((NUM_GRAPHS, c), jnp.float32),
  )(sums, maxs, cnts, wfc, bfc)


def kernel(x, edge_index, batch_index, W1, b1, W2, b2, W3, b3, W4, b4,
           Wfc, bfc):
  n, d = x.shape
  h = W1.shape[1]
  e = edge_index.shape[1]
  g = NUM_GRAPHS

  n_acc = pl.cdiv(n + 1, BLK) * BLK
  k_chunks = pl.cdiv(e, NW * CH)
  k_chunks += k_chunks % 2
  e_pad = NW * k_chunks * CH

  src = edge_index[0].astype(jnp.int32)
  dst = edge_index[1].astype(jnp.int32)
  src_r = jnp.concatenate(
      [src, jnp.full((e_pad - e,), n, jnp.int32)]).reshape(NW, k_chunks, CH)
  dst_r = jnp.concatenate(
      [dst, jnp.full((e_pad - e,), n, jnp.int32)]).reshape(NW, k_chunks, CH)
  x_pad = jnp.pad(x, ((0, n_acc - n), (0, 0)))
  bat_r = jnp.pad(batch_index.astype(jnp.int32), (0, n_acc - n),
                  constant_values=g).reshape(NW, n_acc // NW)

  zero_h = jnp.zeros((n_acc, h), jnp.float32)
  zero_8 = jnp.zeros((n_acc, 8), jnp.float32)
  ones_8 = jnp.ones((CH, 8), jnp.float32)
  neg = jnp.full((g + 1, h), -jnp.inf, jnp.float32)

  dp = _deg_kernel(n_acc, k_chunks)(dst_r, ones_8, zero_8)
  h1p, dinv8 = _tc1(x_pad, W1, dp)

  edge = _edge_kernel(n_acc, h, k_chunks)
  p = edge(h1p, src_r, dst_r, zero_h)
  h2p = _tc_mid(p, h1p, dinv8, b1.reshape(1, h), W2)
  p = edge(h2p, src_r, dst_r, zero_h)
  h3p = _tc_mid(p, h2p, dinv8, b2.reshape(1, h), W3)
  p = edge(h3p, src_r, dst_r, zero_h)
  h4p = _tc_mid(p, h3p, dinv8, b3.reshape(1, h), W4)
  p = edge(h4p, src_r, dst_r, zero_h)
  hidden = _tc_last(p, h4p, dinv8, b4.reshape(1, h))

  sums, maxs, cnts = _pool_kernel(n_acc, h, g)(hidden, bat_r, neg, zero_h)
  out = _tc_final(sums, maxs, cnts, Wfc, bfc.reshape(1, Wfc.shape[1]))
  return (out, hidden[:n])


# speedup vs baseline: 25.1689x; 2.3036x over previous
import functools

import jax
import jax.numpy as jnp
from jax import lax
from jax.experimental import pallas as pl
from jax.experimental.pallas import tpu as pltpu
from jax.experimental.pallas import tpu_sc as plsc

NC, NS = 2, 16
NW = NC * NS
CH = 128
NUM_GRAPHS = 64
BLK = 512


@functools.lru_cache(maxsize=None)
def _deg_kernel(n_acc, k_chunks):
  mesh = plsc.VectorSubcoreMesh(core_axis_name="c", subcore_axis_name="s")
  stripe = n_acc // NS

  @functools.partial(
      pl.kernel,
      mesh=mesh,
      compiler_params=pltpu.CompilerParams(use_tc_tiling_on_sc=False),
      out_type=jax.ShapeDtypeStruct((NC, n_acc, 8), jnp.float32),
      scratch_types=[
          pltpu.VMEM((k_chunks, CH), jnp.int32),
          pltpu.VMEM((CH, 8), jnp.float32),
          pltpu.VMEM_SHARED((n_acc, 8), jnp.float32),
      ],
  )
  def deg(dst_hbm, ones_hbm, zero_hbm, dp_out, dst_v, ones_v, acc):
    c = lax.axis_index("c")
    s = lax.axis_index("s")
    wid = c * NS + s
    pltpu.sync_copy(dst_hbm.at[wid], dst_v)
    pltpu.sync_copy(ones_hbm, ones_v)
    pltpu.sync_copy(zero_hbm.at[pl.ds(s * stripe, stripe)],
                    acc.at[pl.ds(s * stripe, stripe)])
    plsc.subcore_barrier()

    def body(j, carry):
      pltpu.sync_copy(ones_v, acc.at[dst_v.at[j]], add=True)
      return carry

    lax.fori_loop(0, k_chunks, body, 0)
    plsc.subcore_barrier()
    pltpu.sync_copy(acc.at[pl.ds(s * stripe, stripe)],
                    dp_out.at[c, pl.ds(s * stripe, stripe)])

  return deg


@functools.lru_cache(maxsize=None)
def _edge_kernel(n_acc, h, k_chunks):
  mesh = plsc.VectorSubcoreMesh(core_axis_name="c", subcore_axis_name="s")
  stripe = n_acc // NS

  @functools.partial(
      pl.kernel,
      mesh=mesh,
      compiler_params=pltpu.CompilerParams(use_tc_tiling_on_sc=False),
      out_type=jax.ShapeDtypeStruct((NC, n_acc, h), jnp.float32),
      scratch_types=[
          pltpu.VMEM((k_chunks, CH), jnp.int32),
          pltpu.VMEM((k_chunks, CH), jnp.int32),
          pltpu.VMEM((2, CH, h), jnp.float32),
          pltpu.VMEM_SHARED((n_acc, h), jnp.float32),
          pltpu.VMEM_SHARED((n_acc, h), jnp.float32),
          pltpu.SemaphoreType.DMA,
          pltpu.SemaphoreType.DMA,
      ],
  )
  def edge(h_hbm, src_hbm, dst_hbm, zero_hbm, p_out,
           src_v, dst_v, rows_v, acc, h_sp, sem0, sem1):
    c = lax.axis_index("c")
    s = lax.axis_index("s")
    wid = c * NS + s
    pltpu.sync_copy(src_hbm.at[wid], src_v)
    pltpu.sync_copy(dst_hbm.at[wid], dst_v)
    pltpu.sync_copy(zero_hbm.at[pl.ds(s * stripe, stripe)],
                    acc.at[pl.ds(s * stripe, stripe)])
    pltpu.sync_copy(h_hbm.at[pl.ds(s * stripe, stripe)],
                    h_sp.at[pl.ds(s * stripe, stripe)])
    plsc.subcore_barrier()

    sems = (sem0, sem1)

    def gather(j, b):
      return pltpu.make_async_copy(h_sp.at[src_v.at[j]], rows_v.at[b],
                                   sems[b])

    gather(0, 0).start()

    def body(i, carry):
      j0 = 2 * i
      j1 = j0 + 1
      gather(j0, 0).wait()
      gather(j1, 1).start()
      pltpu.sync_copy(rows_v.at[0], acc.at[dst_v.at[j0]], add=True)
      gather(j1, 1).wait()

      @pl.when(i < k_chunks // 2 - 1)
      def _():
        gather(j1 + 1, 0).start()

      pltpu.sync_copy(rows_v.at[1], acc.at[dst_v.at[j1]], add=True)
      return carry

    lax.fori_loop(0, k_chunks // 2, body, 0)
    plsc.subcore_barrier()
    pltpu.sync_copy(acc.at[pl.ds(s * stripe, stripe)],
                    p_out.at[c, pl.ds(s * stripe, stripe)])

  return edge


@functools.lru_cache(maxsize=None)
def _pool_kernel(n_acc, h, g):
  mesh = plsc.VectorSubcoreMesh(core_axis_name="c", subcore_axis_name="s")
  rows = n_acc // NW
  gp = g + 1

  @functools.partial(
      pl.kernel,
      mesh=mesh,
      compiler_params=pltpu.CompilerParams(use_tc_tiling_on_sc=False),
      out_type=(
          jax.ShapeDtypeStruct((NW, gp, h), jnp.float32),
          jax.ShapeDtypeStruct((NW, gp, h), jnp.float32),
          jax.ShapeDtypeStruct((NW, gp, h), jnp.float32),
      ),
      scratch_types=[
          pltpu.VMEM((rows, h), jnp.float32),
          pltpu.VMEM((rows,), jnp.int32),
          pltpu.VMEM((gp, h), jnp.float32),
          pltpu.VMEM((gp, h), jnp.float32),
          pltpu.VMEM((gp, h), jnp.float32),
      ],
  )
  def pool(hid_hbm, bat_hbm, neg_hbm, zero_hbm, sums_o, maxs_o, cnts_o,
           hid_v, bat_v, sum_v, max_v, cnt_v):
    c = lax.axis_index("c")
    s = lax.axis_index("s")
    wid = c * NS + s
    pltpu.sync_copy(hid_hbm.at[pl.ds(wid * rows, rows)], hid_v)
    pltpu.sync_copy(bat_hbm.at[wid], bat_v)
    pltpu.sync_copy(neg_hbm, max_v)
    pltpu.sync_copy(zero_hbm.at[pl.ds(0, gp)], sum_v)
    pltpu.sync_copy(zero_hbm.at[pl.ds(0, gp)], cnt_v)

    def body(q, carry):
      base = q * 16
      bid = bat_v[pl.ds(base, 16)]
      for j in range(16):
        i = base + j
        b = bid[j]
        for k in range(h // 16):
          sl = pl.ds(k * 16, 16)
          r = hid_v[i, sl]
          sum_v[b, sl] = sum_v[b, sl] + r
          max_v[b, sl] = jnp.maximum(max_v[b, sl], r)
        c16 = pl.ds(0, 16)
        cnt_v[b, c16] = cnt_v[b, c16] + 1.0
      return carry

    lax.fori_loop(0, rows // 16, body, 0)
    pltpu.sync_copy(sum_v, sums_o.at[wid])
    pltpu.sync_copy(max_v, maxs_o.at[wid])
    pltpu.sync_copy(cnt_v, cnts_o.at[wid])

  return pool


def _tc1_body(x_ref, w_ref, dp_ref, h_out, dinv_out):
  dp = dp_ref[:]
  dinv = lax.rsqrt(dp[0] + dp[1] + 1.0)
  hm = jnp.dot(x_ref[:], w_ref[:], preferred_element_type=jnp.float32)
  h_out[:] = hm * dinv[:, :1]
  dinv_out[:] = dinv


def _tc1(x_pad, w1, dp):
  n_acc, d = x_pad.shape
  h = w1.shape[1]
  grid = n_acc // BLK
  return pl.pallas_call(
      _tc1_body,
      grid=(grid,),
      in_specs=[
          pl.BlockSpec((BLK, d), lambda i: (i, 0)),
          pl.BlockSpec((d, h), lambda i: (0, 0)),
          pl.BlockSpec((NC, BLK, 8), lambda i: (0, i, 0)),
      ],
      out_specs=[
          pl.BlockSpec((BLK, h), lambda i: (i, 0)),
          pl.BlockSpec((BLK, 8), lambda i: (i, 0)),
      ],
      out_shape=[
          jax.ShapeDtypeStruct((n_acc, h), jnp.float32),
          jax.ShapeDtypeStruct((n_acc, 8), jnp.float32),
      ],
  )(x_pad, w1, dp)


def _tc_mid_body(p_ref, hp_ref, dinv_ref, b_ref, w_ref, out_ref):
  p = p_ref[:]
  d = dinv_ref[:, :1]
  t = jnp.tanh(d * (p[0] + p[1] + hp_ref[:]) + b_ref[:])
  out_ref[:] = jnp.dot(t, w_ref[:], preferred_element_type=jnp.float32) * d


def _tc_mid(p, hp, dinv8, b, w):
  n_acc, h = hp.shape
  grid = n_acc // BLK
  return pl.pallas_call(
      _tc_mid_body,
      grid=(grid,),
      in_specs=[
          pl.BlockSpec((NC, BLK, h), lambda i: (0, i, 0)),
          pl.BlockSpec((BLK, h), lambda i: (i, 0)),
          pl.BlockSpec((BLK, 8), lambda i: (i, 0)),
          pl.BlockSpec((1, h), lambda i: (0, 0)),
          pl.BlockSpec((h, h), lambda i: (0, 0)),
      ],
      out_specs=pl.BlockSpec((BLK, h), lambda i: (i, 0)),
      out_shape=jax.ShapeDtypeStruct((n_acc, h), jnp.float32),
  )(p, hp, dinv8, b, w)


def _tc_last_body(p_ref, hp_ref, dinv_ref, b_ref, out_ref):
  p = p_ref[:]
  d = dinv_ref[:, :1]
  out_ref[:] = jnp.tanh(d * (p[0] + p[1] + hp_ref[:]) + b_ref[:])


def _tc_last(p, hp, dinv8, b):
  n_acc, h = hp.shape
  grid = n_acc // BLK
  return pl.pallas_call(
      _tc_last_body,
      grid=(grid,),
      in_specs=[
          pl.BlockSpec((NC, BLK, h), lambda i: (0, i, 0)),
          pl.BlockSpec((BLK, h), lambda i: (i, 0)),
          pl.BlockSpec((BLK, 8), lambda i: (i, 0)),
          pl.BlockSpec((1, h), lambda i: (0, 0)),
      ],
      out_specs=pl.BlockSpec((BLK, h), lambda i: (i, 0)),
      out_shape=jax.ShapeDtypeStruct((n_acc, h), jnp.float32),
  )(p, hp, dinv8, b)


def _tc_final_body(sums_ref, maxs_ref, cnts_ref, wfc_ref, bfc_ref, out_ref):
  g = NUM_GRAPHS
  s = jnp.sum(sums_ref[:], axis=0)[:g]
  m = jnp.max(maxs_ref[:], axis=0)[:g]
  cnt = jnp.sum(cnts_ref[:], axis=0)[:g, :1]
  gmean = s / jnp.maximum(cnt, 1.0)
  pooled = jnp.concatenate([m, gmean], axis=1)
  out_ref[:] = (
      jnp.dot(pooled, wfc_ref[:], preferred_element_type=jnp.float32)
      + bfc_ref[:])


def _tc_final(sums, maxs, cnts, wfc, bfc):
  c = wfc.shape[1]
  return pl.pallas_call(
      _tc_final_body,
      out_shape=jax.ShapeDtypeStruct((NUM_GRAPHS, c), jnp.float32),
  )(sums, maxs, cnts, wfc, bfc)


def kernel(x, edge_index, batch_index, W1, b1, W2, b2, W3, b3, W4, b4,
           Wfc, bfc):
  n, d = x.shape
  h = W1.shape[1]
  e = edge_index.shape[1]
  g = NUM_GRAPHS

  n_acc = pl.cdiv(n + 1, BLK) * BLK
  k_chunks = pl.cdiv(e, NW * CH)
  k_chunks += k_chunks % 2
  e_pad = NW * k_chunks * CH

  src = edge_index[0].astype(jnp.int32)
  dst = edge_index[1].astype(jnp.int32)
  src_r = jnp.concatenate(
      [src, jnp.full((e_pad - e,), n, jnp.int32)]).reshape(NW, k_chunks, CH)
  dst_r = jnp.concatenate(
      [dst, jnp.full((e_pad - e,), n, jnp.int32)]).reshape(NW, k_chunks, CH)
  x_pad = jnp.pad(x, ((0, n_acc - n), (0, 0)))
  bat_r = jnp.pad(batch_index.astype(jnp.int32), (0, n_acc - n),
                  constant_values=g).reshape(NW, n_acc // NW)

  zero_h = jnp.zeros((n_acc, h), jnp.float32)
  zero_8 = jnp.zeros((n_acc, 8), jnp.float32)
  ones_8 = jnp.ones((CH, 8), jnp.float32)
  neg = jnp.full((g + 1, h), -jnp.inf, jnp.float32)

  dp = _deg_kernel(n_acc, k_chunks)(dst_r, ones_8, zero_8)
  h1p, dinv8 = _tc1(x_pad, W1, dp)

  edge = _edge_kernel(n_acc, h, k_chunks)
  p = edge(h1p, src_r, dst_r, zero_h)
  h2p = _tc_mid(p, h1p, dinv8, b1.reshape(1, h), W2)
  p = edge(h2p, src_r, dst_r, zero_h)
  h3p = _tc_mid(p, h2p, dinv8, b2.reshape(1, h), W3)
  p = edge(h3p, src_r, dst_r, zero_h)
  h4p = _tc_mid(p, h3p, dinv8, b3.reshape(1, h), W4)
  p = edge(h4p, src_r, dst_r, zero_h)
  hidden = _tc_last(p, h4p, dinv8, b4.reshape(1, h))

  sums, maxs, cnts = _pool_kernel(n_acc, h, g)(hidden, bat_r, neg, zero_h)
  out = _tc_final(sums, maxs, cnts, Wfc, bfc.reshape(1, Wfc.shape[1]))
  return (out, hidden[:n])
